# baseline (speedup 1.0000x reference)
import functools

import jax
import jax.numpy as jnp
from jax import lax
from jax.experimental import pallas as pl
from jax.experimental.pallas import tpu as pltpu
from jax.experimental.pallas import tpu_sc as plsc

N = 10000
H = 128
SEQ = 8
NP = 10064
NT = 32
NSC = 2
NSUB = 16
CHUNK = 128
SUBROWS = 624
TAIL = NP - NSUB * SUBROWS
NDUM = 64
_RB = 5032


def _mesh():
  return plsc.VectorSubcoreMesh(
      core_axis_name="c", subcore_axis_name="s",
      num_cores=NSC, num_subcores=NSUB)


def _make_degrees(span):
  vecs = span // 16

  @functools.partial(
      pl.kernel,
      mesh=_mesh(),
      compiler_params=pltpu.CompilerParams(needs_layout_passes=False),
      out_type=jax.ShapeDtypeStruct((2, NT, NP), jnp.float32),
      scratch_types=[
          pltpu.VMEM((span,), jnp.int32),
          pltpu.VMEM((span,), jnp.int32),
          pltpu.VMEM((NP,), jnp.float32),
          pltpu.VMEM((NP,), jnp.float32),
      ],
  )
  def deg_kernel(src_hbm, dst_hbm, out_hbm, src_v, dst_v, hs_v, hd_v):
    c = lax.axis_index("c")
    s = lax.axis_index("s")
    wid = c * NSUB + s
    zeros16 = jnp.zeros((16,), jnp.float32)
    ones16 = jnp.ones((16,), jnp.float32)

    def zero_body(k, carry):
      hs_v[pl.ds(k * 16, 16)] = zeros16
      hd_v[pl.ds(k * 16, 16)] = zeros16
      return carry

    lax.fori_loop(0, NP // 16, zero_body, 0, unroll=8)

    pltpu.sync_copy(src_hbm.at[wid], src_v)
    pltpu.sync_copy(dst_hbm.at[wid], dst_v)

    def hist_body(k, carry):
      si = src_v[pl.ds(k * 16, 16)]
      di = dst_v[pl.ds(k * 16, 16)]
      plsc.addupdate_scatter(hs_v, [si], ones16)
      plsc.addupdate_scatter(hd_v, [di], ones16)
      return carry

    lax.fori_loop(0, vecs, hist_body, 0, unroll=8)

    pltpu.sync_copy(hs_v, out_hbm.at[0, wid])
    pltpu.sync_copy(hd_v, out_hbm.at[1, wid])

  return deg_kernel


def _make_wsum(span):
  vecs = span // 16

  @functools.partial(
      pl.kernel,
      mesh=_mesh(),
      compiler_params=pltpu.CompilerParams(needs_layout_passes=False),
      out_type=jax.ShapeDtypeStruct((NT, NP), jnp.float32),
      scratch_types=[
          pltpu.VMEM((NP,), jnp.float32),
          pltpu.VMEM((span,), jnp.int32),
          pltpu.VMEM((span,), jnp.int32),
          pltpu.VMEM((NP,), jnp.float32),
      ],
  )
  def wsum_kernel(invout_hbm, src_hbm, dst_hbm, out_hbm,
                  inv_v, src_v, dst_v, hist_v):
    c = lax.axis_index("c")
    s = lax.axis_index("s")
    wid = c * NSUB + s
    zeros16 = jnp.zeros((16,), jnp.float32)

    def zero_body(k, carry):
      hist_v[pl.ds(k * 16, 16)] = zeros16
      return carry

    lax.fori_loop(0, NP // 16, zero_body, 0, unroll=8)

    pltpu.sync_copy(invout_hbm, inv_v)
    pltpu.sync_copy(src_hbm.at[wid], src_v)
    pltpu.sync_copy(dst_hbm.at[wid], dst_v)

    def hist_body(k, carry):
      si = src_v[pl.ds(k * 16, 16)]
      vals = plsc.load_gather(inv_v, [si])
      di = dst_v[pl.ds(k * 16, 16)]
      plsc.addupdate_scatter(hist_v, [di], vals)
      return carry

    lax.fori_loop(0, vecs, hist_body, 0, unroll=8)

    pltpu.sync_copy(hist_v, out_hbm.at[wid])

  return wsum_kernel


def _make_spmm(nch):
  ndep = 3
  assert nch % ndep == 0

  @functools.partial(
      pl.kernel,
      mesh=_mesh(),
      compiler_params=pltpu.CompilerParams(needs_layout_passes=False),
      out_type=jax.ShapeDtypeStruct((NSC, NP, H), jnp.float32),
      scratch_types=[
          pltpu.VMEM((ndep, 2, CHUNK), jnp.int32),
          [pltpu.VMEM((CHUNK, H), jnp.float32) for _ in range(ndep)],
          pltpu.VMEM_SHARED((NP, H), jnp.float32),
          [pltpu.SemaphoreType.DMA for _ in range(ndep)],
          [pltpu.SemaphoreType.DMA for _ in range(ndep)],
      ],
  )
  def spmm_kernel(hs_hbm, edge_hbm, out_hbm,
                  idxring, bufs, agg_sh, isems, dsems):
    c = lax.axis_index("c")
    s = lax.axis_index("s")
    wid = c * NSUB + s
    zeros16 = jnp.zeros((16,), jnp.float32)

    def zb(k, carry):
      bufs[0][k // (H // 16), pl.ds((k % (H // 16)) * 16, 16)] = zeros16
      return carry

    lax.fori_loop(0, CHUNK * (H // 16), zb, 0)
    base = s * SUBROWS

    def zs(t, carry):
      pltpu.sync_copy(bufs[0].at[pl.ds(0, CHUNK)],
                      agg_sh.at[pl.ds(base + t * CHUNK, CHUNK)])
      return carry

    lax.fori_loop(0, SUBROWS // CHUNK, zs, 0)
    rem = SUBROWS % CHUNK
    if rem:
      pltpu.sync_copy(
          bufs[0].at[pl.ds(0, rem)],
          agg_sh.at[pl.ds(base + SUBROWS - rem, rem)])

    @pl.when(s == 0)
    def _():
      pltpu.sync_copy(bufs[0].at[pl.ds(0, TAIL)],
                      agg_sh.at[pl.ds(NSUB * SUBROWS, TAIL)])

    plsc.subcore_barrier()

    def idx_cp(k, slot):
      return pltpu.make_async_copy(edge_hbm.at[wid, k], idxring.at[slot],
                                   isems[slot])

    def gat_cp(slot):
      return pltpu.make_async_copy(hs_hbm.at[idxring.at[slot, 0]],
                                   bufs[slot], dsems[slot])

    idx_cp(0, 0).start()
    idx_cp(1, 1).start()
    for u in range(2):
      idx_cp(u, u).wait()
      gat_cp(u).start()
    idx_cp(2, 2).start()

    def step(g, carry):
      for u in range(ndep):
        j = g * ndep + u
        gat_cp(u).wait()

        @pl.when(j + 2 < nch)
        def _():
          idx_cp(j + 2, (u + 2) % ndep).wait()
          gat_cp((u + 2) % ndep).start()

        pltpu.sync_copy(bufs[u], agg_sh.at[idxring.at[u, 1]], add=True)
        pltpu.sync_copy(bufs[u], agg_sh.at[idxring.at[u, 1]], add=True)

        @pl.when(j + 3 < nch)
        def _():
          idx_cp(j + 3, u).start()
      return carry

    lax.fori_loop(0, nch // ndep, step, 0)
    plsc.subcore_barrier()

    pltpu.sync_copy(
        agg_sh.at[pl.ds(s * SUBROWS, SUBROWS)],
        out_hbm.at[c, pl.ds(s * SUBROWS, SUBROWS)])

    @pl.when(s == 0)
    def _():
      pltpu.sync_copy(
          agg_sh.at[pl.ds(NSUB * SUBROWS, TAIL)],
          out_hbm.at[c, pl.ds(NSUB * SUBROWS, TAIL)])

  return spmm_kernel


def _precompute_body(degp_ref, x_ref, wr_ref, wz_ref, wh_ref, bias_ref,
                     consts_ref, invout_ref, invin_ref):
  deg = jnp.sum(degp_ref[...], axis=1)
  inv = jnp.where(deg > 0, lax.rsqrt(deg), 0.0)
  invout_ref[...] = inv[0][:, None]
  invin_ref[...] = inv[1][:, None]

  x = x_ref[...]
  xr = jnp.dot(x, wr_ref[...], preferred_element_type=jnp.float32)
  xz = jnp.dot(x, wz_ref[...], preferred_element_type=jnp.float32)
  xh = jnp.dot(x, wh_ref[...], preferred_element_type=jnp.float32)
  proj = jnp.concatenate(
      [xr, xz, xh, jnp.zeros((5, H), jnp.float32)], axis=0)
  consts_ref[...] = proj + bias_ref[...]


def _precompute(degp, x, wr, wz, wh, bias_pack):
  return pl.pallas_call(
      _precompute_body,
      out_shape=[
          jax.ShapeDtypeStruct((8, H), jnp.float32),
          jax.ShapeDtypeStruct((NP, 1), jnp.float32),
          jax.ShapeDtypeStruct((NP, 1), jnp.float32),
      ],
  )(degp, x, wr, wz, wh, bias_pack)


def _gru_tail(agg, h_ref, invout_ref, c_ref, w_ref, hn_ref, hs_ref):
  gh = jnp.dot(agg, w_ref[...], preferred_element_type=jnp.float32)
  gh = gh + c_ref[3:4]
  r = jax.nn.sigmoid(c_ref[0:1] + gh)
  z = jax.nn.sigmoid(c_ref[1:2] + gh)
  ht = jnp.tanh(c_ref[2:3] + r * gh)
  hn = (1.0 - z) * h_ref[...] + z * ht
  hn_ref[...] = hn
  hs_ref[...] = hn * invout_ref[...]


def _step_body(p_ref, h_ref, invin_ref, invout_ref, c_ref, w_ref,
               hn_ref, hs_ref):
  agg = (p_ref[0] + p_ref[1]) * invin_ref[...]
  _gru_tail(agg, h_ref, invout_ref, c_ref, w_ref, hn_ref, hs_ref)


def _step2_body(s_ref, h_ref, invin_ref, invout_ref, c_ref, w_ref,
                hn_ref, hs_ref):
  agg = h_ref[...] * (invin_ref[...] * s_ref[...])
  _gru_tail(agg, h_ref, invout_ref, c_ref, w_ref, hn_ref, hs_ref)


def _reduce_wsum_body(wsum_ref, s_ref):
  s_ref[...] = jnp.sum(wsum_ref[...], axis=0)[:, None]


def _reduce_wsum(wsump):
  return pl.pallas_call(
      _reduce_wsum_body,
      out_shape=jax.ShapeDtypeStruct((NP, 1), jnp.float32),
  )(wsump)


def _tc_step(p, h, invin, invout, consts, gcn_W):
  grid = (NP // _RB,)
  return pl.pallas_call(
      _step_body,
      grid=grid,
      in_specs=[
          pl.BlockSpec((NSC, _RB, H), lambda j: (0, j, 0)),
          pl.BlockSpec((_RB, H), lambda j: (j, 0)),
          pl.BlockSpec((_RB, 1), lambda j: (j, 0)),
          pl.BlockSpec((_RB, 1), lambda j: (j, 0)),
          pl.BlockSpec((8, H), lambda j: (0, 0)),
          pl.BlockSpec((H, H), lambda j: (0, 0)),
      ],
      out_specs=[
          pl.BlockSpec((_RB, H), lambda j: (j, 0)),
          pl.BlockSpec((_RB, H), lambda j: (j, 0)),
      ],
      out_shape=[
          jax.ShapeDtypeStruct((NP, H), jnp.float32),
          jax.ShapeDtypeStruct((NP, H), jnp.float32),
      ],
  )(p, h, invin, invout, consts, gcn_W)


def _tc_step2(svec, h, invin, invout, consts, gcn_W):
  grid = (NP // _RB,)
  return pl.pallas_call(
      _step2_body,
      grid=grid,
      in_specs=[
          pl.BlockSpec((_RB, 1), lambda j: (j, 0)),
          pl.BlockSpec((_RB, H), lambda j: (j, 0)),
          pl.BlockSpec((_RB, 1), lambda j: (j, 0)),
          pl.BlockSpec((_RB, 1), lambda j: (j, 0)),
          pl.BlockSpec((8, H), lambda j: (0, 0)),
          pl.BlockSpec((H, H), lambda j: (0, 0)),
      ],
      out_specs=[
          pl.BlockSpec((_RB, H), lambda j: (j, 0)),
          pl.BlockSpec((_RB, H), lambda j: (j, 0)),
      ],
      out_shape=[
          jax.ShapeDtypeStruct((NP, H), jnp.float32),
          jax.ShapeDtypeStruct((NP, H), jnp.float32),
      ],
  )(svec, h, invin, invout, consts, gcn_W)


def kernel(x, edge_index, w_r_W, w_r_b, w_z_W, w_z_b, w_h_W, w_h_b,
           gcn_W, gcn_b):
  E = edge_index.shape[1]
  span = -(-E // NT)
  span = -(-span // (3 * CHUNK)) * (3 * CHUNK)
  ep = NT * span
  pad = ep - E
  nch = span // CHUNK

  src = edge_index[0]
  dst = edge_index[1]
  if pad:
    fill = N + (jnp.arange(pad, dtype=jnp.int32) % NDUM)
    src = jnp.concatenate([src, fill])
    dst = jnp.concatenate([dst, fill])
  src_f = src.reshape(NT, span)
  dst_f = dst.reshape(NT, span)
  src_t = src.reshape(NT, nch, CHUNK)
  dst_t = dst.reshape(NT, nch, CHUNK)
  edge_t = jnp.stack([src_t, dst_t], axis=2)

  degp = _make_degrees(span)(src_f, dst_f)

  bias_pack = jnp.zeros((8, H), jnp.float32)
  bias_pack = bias_pack.at[0].set(w_r_b).at[1].set(w_z_b)
  bias_pack = bias_pack.at[2].set(w_h_b).at[3].set(gcn_b)

  consts, invout, invin = _precompute(
      degp, x.reshape(1, H), w_r_W, w_z_W, w_h_W, bias_pack)

  svec = _reduce_wsum(_make_wsum(span)(invout.reshape(NP), src_f, dst_f))

  spmm = _make_spmm(nch)
  h = jnp.zeros((NP, H), jnp.float32)
  outs = []
  h, _ = _tc_step(jnp.zeros((NSC, NP, H), jnp.float32), h,
                  invin, invout, consts, gcn_W)
  outs.append(h[:N])
  h, hs = _tc_step2(svec, h, invin, invout, consts, gcn_W)
  outs.append(h[:N])
  for _ in range(2, SEQ):
    p = spmm(hs, edge_t)
    h, hs = _tc_step(p, h, invin, invout, consts, gcn_W)
    outs.append(h[:N])
  return jnp.stack(outs, axis=0)[None]


# speedup vs baseline: 1.3139x; 1.3139x over previous
import functools

import jax
import jax.numpy as jnp
from jax import lax
from jax.experimental import pallas as pl
from jax.experimental.pallas import tpu as pltpu
from jax.experimental.pallas import tpu_sc as plsc

N = 10000
H = 128
SEQ = 8
NP = 10064
NT = 32
NSC = 2
NSUB = 16
CHUNK = 128
SUBROWS = 624
TAIL = NP - NSUB * SUBROWS
NDUM = 64
_RB = 5032


def _mesh():
  return plsc.VectorSubcoreMesh(
      core_axis_name="c", subcore_axis_name="s",
      num_cores=NSC, num_subcores=NSUB)


def _make_degrees(span):
  vecs = span // 16

  @functools.partial(
      pl.kernel,
      mesh=_mesh(),
      compiler_params=pltpu.CompilerParams(needs_layout_passes=False),
      out_type=jax.ShapeDtypeStruct((2, NT, NP), jnp.float32),
      scratch_types=[
          pltpu.VMEM((span,), jnp.int32),
          pltpu.VMEM((span,), jnp.int32),
          pltpu.VMEM((NP,), jnp.float32),
          pltpu.VMEM((NP,), jnp.float32),
      ],
  )
  def deg_kernel(src_hbm, dst_hbm, out_hbm, src_v, dst_v, hs_v, hd_v):
    c = lax.axis_index("c")
    s = lax.axis_index("s")
    wid = c * NSUB + s
    zeros16 = jnp.zeros((16,), jnp.float32)
    ones16 = jnp.ones((16,), jnp.float32)

    def zero_body(k, carry):
      hs_v[pl.ds(k * 16, 16)] = zeros16
      hd_v[pl.ds(k * 16, 16)] = zeros16
      return carry

    lax.fori_loop(0, NP // 16, zero_body, 0, unroll=8)

    pltpu.sync_copy(src_hbm.at[wid], src_v)
    pltpu.sync_copy(dst_hbm.at[wid], dst_v)

    def hist_body(k, carry):
      si = src_v[pl.ds(k * 16, 16)]
      di = dst_v[pl.ds(k * 16, 16)]
      plsc.addupdate_scatter(hs_v, [si], ones16)
      plsc.addupdate_scatter(hd_v, [di], ones16)
      return carry

    lax.fori_loop(0, vecs, hist_body, 0, unroll=8)

    pltpu.sync_copy(hs_v, out_hbm.at[0, wid])
    pltpu.sync_copy(hd_v, out_hbm.at[1, wid])

  return deg_kernel


def _make_wsum(span):
  vecs = span // 16

  @functools.partial(
      pl.kernel,
      mesh=_mesh(),
      compiler_params=pltpu.CompilerParams(needs_layout_passes=False),
      out_type=jax.ShapeDtypeStruct((NT, NP), jnp.float32),
      scratch_types=[
          pltpu.VMEM((NP,), jnp.float32),
          pltpu.VMEM((span,), jnp.int32),
          pltpu.VMEM((span,), jnp.int32),
          pltpu.VMEM((NP,), jnp.float32),
      ],
  )
  def wsum_kernel(invout_hbm, src_hbm, dst_hbm, out_hbm,
                  inv_v, src_v, dst_v, hist_v):
    c = lax.axis_index("c")
    s = lax.axis_index("s")
    wid = c * NSUB + s
    zeros16 = jnp.zeros((16,), jnp.float32)

    def zero_body(k, carry):
      hist_v[pl.ds(k * 16, 16)] = zeros16
      return carry

    lax.fori_loop(0, NP // 16, zero_body, 0, unroll=8)

    pltpu.sync_copy(invout_hbm, inv_v)
    pltpu.sync_copy(src_hbm.at[wid], src_v)
    pltpu.sync_copy(dst_hbm.at[wid], dst_v)

    def hist_body(k, carry):
      si = src_v[pl.ds(k * 16, 16)]
      vals = plsc.load_gather(inv_v, [si])
      di = dst_v[pl.ds(k * 16, 16)]
      plsc.addupdate_scatter(hist_v, [di], vals)
      return carry

    lax.fori_loop(0, vecs, hist_body, 0, unroll=8)

    pltpu.sync_copy(hist_v, out_hbm.at[wid])

  return wsum_kernel


def _make_spmm(nch):
  ndep = 3
  assert nch % ndep == 0

  @functools.partial(
      pl.kernel,
      mesh=_mesh(),
      compiler_params=pltpu.CompilerParams(needs_layout_passes=False),
      out_type=jax.ShapeDtypeStruct((NSC, NP, H), jnp.float32),
      scratch_types=[
          pltpu.VMEM((ndep, 2, CHUNK), jnp.int32),
          [pltpu.VMEM((CHUNK, H), jnp.float32) for _ in range(ndep)],
          pltpu.VMEM_SHARED((NP, H), jnp.float32),
          [pltpu.SemaphoreType.DMA for _ in range(ndep)],
          [pltpu.SemaphoreType.DMA for _ in range(ndep)],
      ],
  )
  def spmm_kernel(hs_hbm, edge_hbm, out_hbm,
                  idxring, bufs, agg_sh, isems, dsems):
    c = lax.axis_index("c")
    s = lax.axis_index("s")
    wid = c * NSUB + s
    zeros16 = jnp.zeros((16,), jnp.float32)

    def zb(k, carry):
      bufs[0][k // (H // 16), pl.ds((k % (H // 16)) * 16, 16)] = zeros16
      return carry

    lax.fori_loop(0, CHUNK * (H // 16), zb, 0)
    base = s * SUBROWS

    def zs(t, carry):
      pltpu.sync_copy(bufs[0].at[pl.ds(0, CHUNK)],
                      agg_sh.at[pl.ds(base + t * CHUNK, CHUNK)])
      return carry

    lax.fori_loop(0, SUBROWS // CHUNK, zs, 0)
    rem = SUBROWS % CHUNK
    if rem:
      pltpu.sync_copy(
          bufs[0].at[pl.ds(0, rem)],
          agg_sh.at[pl.ds(base + SUBROWS - rem, rem)])

    @pl.when(s == 0)
    def _():
      pltpu.sync_copy(bufs[0].at[pl.ds(0, TAIL)],
                      agg_sh.at[pl.ds(NSUB * SUBROWS, TAIL)])

    plsc.subcore_barrier()

    def idx_cp(k, slot):
      return pltpu.make_async_copy(edge_hbm.at[wid, k], idxring.at[slot],
                                   isems[slot])

    def gat_cp(slot):
      return pltpu.make_async_copy(hs_hbm.at[idxring.at[slot, 0]],
                                   bufs[slot], dsems[slot])

    idx_cp(0, 0).start()
    idx_cp(1, 1).start()
    for u in range(2):
      idx_cp(u, u).wait()
      gat_cp(u).start()
    idx_cp(2, 2).start()

    def step(g, carry):
      for u in range(ndep):
        j = g * ndep + u
        gat_cp(u).wait()

        @pl.when(j + 2 < nch)
        def _():
          idx_cp(j + 2, (u + 2) % ndep).wait()
          gat_cp((u + 2) % ndep).start()

        pltpu.sync_copy(bufs[u], agg_sh.at[idxring.at[u, 1]], add=True)

        @pl.when(j + 3 < nch)
        def _():
          idx_cp(j + 3, u).start()
      return carry

    lax.fori_loop(0, nch // ndep, step, 0)
    plsc.subcore_barrier()

    pltpu.sync_copy(
        agg_sh.at[pl.ds(s * SUBROWS, SUBROWS)],
        out_hbm.at[c, pl.ds(s * SUBROWS, SUBROWS)])

    @pl.when(s == 0)
    def _():
      pltpu.sync_copy(
          agg_sh.at[pl.ds(NSUB * SUBROWS, TAIL)],
          out_hbm.at[c, pl.ds(NSUB * SUBROWS, TAIL)])

  return spmm_kernel


def _precompute_body(degp_ref, x_ref, wr_ref, wz_ref, wh_ref, bias_ref,
                     consts_ref, invout_ref, invin_ref):
  deg = jnp.sum(degp_ref[...], axis=1)
  inv = jnp.where(deg > 0, lax.rsqrt(deg), 0.0)
  invout_ref[...] = inv[0][:, None]
  invin_ref[...] = inv[1][:, None]

  x = x_ref[...]
  xr = jnp.dot(x, wr_ref[...], preferred_element_type=jnp.float32)
  xz = jnp.dot(x, wz_ref[...], preferred_element_type=jnp.float32)
  xh = jnp.dot(x, wh_ref[...], preferred_element_type=jnp.float32)
  proj = jnp.concatenate(
      [xr, xz, xh, jnp.zeros((5, H), jnp.float32)], axis=0)
  consts_ref[...] = proj + bias_ref[...]


def _precompute(degp, x, wr, wz, wh, bias_pack):
  return pl.pallas_call(
      _precompute_body,
      out_shape=[
          jax.ShapeDtypeStruct((8, H), jnp.float32),
          jax.ShapeDtypeStruct((NP, 1), jnp.float32),
          jax.ShapeDtypeStruct((NP, 1), jnp.float32),
      ],
  )(degp, x, wr, wz, wh, bias_pack)


def _gru_tail(agg, h_ref, invout_ref, c_ref, w_ref, hn_ref, hs_ref):
  gh = jnp.dot(agg, w_ref[...], preferred_element_type=jnp.float32)
  gh = gh + c_ref[3:4]
  r = jax.nn.sigmoid(c_ref[0:1] + gh)
  z = jax.nn.sigmoid(c_ref[1:2] + gh)
  ht = jnp.tanh(c_ref[2:3] + r * gh)
  hn = (1.0 - z) * h_ref[...] + z * ht
  hn_ref[...] = hn
  hs_ref[...] = hn * invout_ref[...]


def _step_body(p_ref, h_ref, invin_ref, invout_ref, c_ref, w_ref,
               hn_ref, hs_ref):
  agg = (p_ref[0] + p_ref[1]) * invin_ref[...]
  _gru_tail(agg, h_ref, invout_ref, c_ref, w_ref, hn_ref, hs_ref)


def _step2_body(s_ref, h_ref, invin_ref, invout_ref, c_ref, w_ref,
                hn_ref, hs_ref):
  agg = h_ref[...] * (invin_ref[...] * s_ref[...])
  _gru_tail(agg, h_ref, invout_ref, c_ref, w_ref, hn_ref, hs_ref)


def _reduce_wsum_body(wsum_ref, s_ref):
  s_ref[...] = jnp.sum(wsum_ref[...], axis=0)[:, None]


def _reduce_wsum(wsump):
  return pl.pallas_call(
      _reduce_wsum_body,
      out_shape=jax.ShapeDtypeStruct((NP, 1), jnp.float32),
  )(wsump)


def _tc_step(p, h, invin, invout, consts, gcn_W):
  grid = (NP // _RB,)
  return pl.pallas_call(
      _step_body,
      grid=grid,
      in_specs=[
          pl.BlockSpec((NSC, _RB, H), lambda j: (0, j, 0)),
          pl.BlockSpec((_RB, H), lambda j: (j, 0)),
          pl.BlockSpec((_RB, 1), lambda j: (j, 0)),
          pl.BlockSpec((_RB, 1), lambda j: (j, 0)),
          pl.BlockSpec((8, H), lambda j: (0, 0)),
          pl.BlockSpec((H, H), lambda j: (0, 0)),
      ],
      out_specs=[
          pl.BlockSpec((_RB, H), lambda j: (j, 0)),
          pl.BlockSpec((_RB, H), lambda j: (j, 0)),
      ],
      out_shape=[
          jax.ShapeDtypeStruct((NP, H), jnp.float32),
          jax.ShapeDtypeStruct((NP, H), jnp.float32),
      ],
  )(p, h, invin, invout, consts, gcn_W)


def _tc_step2(svec, h, invin, invout, consts, gcn_W):
  grid = (NP // _RB,)
  return pl.pallas_call(
      _step2_body,
      grid=grid,
      in_specs=[
          pl.BlockSpec((_RB, 1), lambda j: (j, 0)),
          pl.BlockSpec((_RB, H), lambda j: (j, 0)),
          pl.BlockSpec((_RB, 1), lambda j: (j, 0)),
          pl.BlockSpec((_RB, 1), lambda j: (j, 0)),
          pl.BlockSpec((8, H), lambda j: (0, 0)),
          pl.BlockSpec((H, H), lambda j: (0, 0)),
      ],
      out_specs=[
          pl.BlockSpec((_RB, H), lambda j: (j, 0)),
          pl.BlockSpec((_RB, H), lambda j: (j, 0)),
      ],
      out_shape=[
          jax.ShapeDtypeStruct((NP, H), jnp.float32),
          jax.ShapeDtypeStruct((NP, H), jnp.float32),
      ],
  )(svec, h, invin, invout, consts, gcn_W)


def kernel(x, edge_index, w_r_W, w_r_b, w_z_W, w_z_b, w_h_W, w_h_b,
           gcn_W, gcn_b):
  E = edge_index.shape[1]
  span = -(-E // NT)
  span = -(-span // (3 * CHUNK)) * (3 * CHUNK)
  ep = NT * span
  pad = ep - E
  nch = span // CHUNK

  src = edge_index[0]
  dst = edge_index[1]
  if pad:
    fill = N + (jnp.arange(pad, dtype=jnp.int32) % NDUM)
    src = jnp.concatenate([src, fill])
    dst = jnp.concatenate([dst, fill])
  src_f = src.reshape(NT, span)
  dst_f = dst.reshape(NT, span)
  src_t = src.reshape(NT, nch, CHUNK)
  dst_t = dst.reshape(NT, nch, CHUNK)
  edge_t = jnp.stack([src_t, dst_t], axis=2)

  degp = _make_degrees(span)(src_f, dst_f)

  bias_pack = jnp.zeros((8, H), jnp.float32)
  bias_pack = bias_pack.at[0].set(w_r_b).at[1].set(w_z_b)
  bias_pack = bias_pack.at[2].set(w_h_b).at[3].set(gcn_b)

  consts, invout, invin = _precompute(
      degp, x.reshape(1, H), w_r_W, w_z_W, w_h_W, bias_pack)

  svec = _reduce_wsum(_make_wsum(span)(invout.reshape(NP), src_f, dst_f))

  spmm = _make_spmm(nch)
  h = jnp.zeros((NP, H), jnp.float32)
  out = jnp.zeros((SEQ, N, H), jnp.float32)
  h, _ = _tc_step2(jnp.zeros((NP, 1), jnp.float32), h,
                   invin, invout, consts, gcn_W)
  out = out.at[0].set(h[:N])
  h, hs = _tc_step2(svec, h, invin, invout, consts, gcn_W)
  out = out.at[1].set(h[:N])
  for t in range(2, SEQ):
    p = spmm(hs, edge_t)
    h, hs = _tc_step(p, h, invin, invout, consts, gcn_W)
    out = out.at[t].set(h[:N])
  return out[None]


# speedup vs baseline: 1.3460x; 1.0244x over previous
import functools

import jax
import jax.numpy as jnp
from jax import lax
from jax.experimental import pallas as pl
from jax.experimental.pallas import tpu as pltpu
from jax.experimental.pallas import tpu_sc as plsc

N = 10000
H = 128
SEQ = 8
NP = 10064
NT = 32
NSC = 2
NSUB = 16
CHUNK = 128
SUBROWS = 624
TAIL = NP - NSUB * SUBROWS
NDUM = 64
_RB = 5032


def _mesh():
  return plsc.VectorSubcoreMesh(
      core_axis_name="c", subcore_axis_name="s",
      num_cores=NSC, num_subcores=NSUB)


def _make_degrees(span):
  vecs = span // 16

  @functools.partial(
      pl.kernel,
      mesh=_mesh(),
      compiler_params=pltpu.CompilerParams(needs_layout_passes=False),
      out_type=jax.ShapeDtypeStruct((2, NT, NP), jnp.float32),
      scratch_types=[
          pltpu.VMEM((span,), jnp.int32),
          pltpu.VMEM((span,), jnp.int32),
          pltpu.VMEM((NP,), jnp.float32),
          pltpu.VMEM((NP,), jnp.float32),
      ],
  )
  def deg_kernel(src_hbm, dst_hbm, out_hbm, src_v, dst_v, hs_v, hd_v):
    c = lax.axis_index("c")
    s = lax.axis_index("s")
    wid = c * NSUB + s
    zeros16 = jnp.zeros((16,), jnp.float32)
    ones16 = jnp.ones((16,), jnp.float32)

    def zero_body(k, carry):
      hs_v[pl.ds(k * 16, 16)] = zeros16
      hd_v[pl.ds(k * 16, 16)] = zeros16
      return carry

    lax.fori_loop(0, NP // 16, zero_body, 0, unroll=8)

    pltpu.sync_copy(src_hbm.at[wid], src_v)
    pltpu.sync_copy(dst_hbm.at[wid], dst_v)

    def hist_body(k, carry):
      si = src_v[pl.ds(k * 16, 16)]
      di = dst_v[pl.ds(k * 16, 16)]
      plsc.addupdate_scatter(hs_v, [si], ones16)
      plsc.addupdate_scatter(hd_v, [di], ones16)
      return carry

    lax.fori_loop(0, vecs, hist_body, 0, unroll=8)

    pltpu.sync_copy(hs_v, out_hbm.at[0, wid])
    pltpu.sync_copy(hd_v, out_hbm.at[1, wid])

  return deg_kernel


def _make_wsum(span):
  vecs = span // 16

  @functools.partial(
      pl.kernel,
      mesh=_mesh(),
      compiler_params=pltpu.CompilerParams(needs_layout_passes=False),
      out_type=jax.ShapeDtypeStruct((NT, NP), jnp.float32),
      scratch_types=[
          pltpu.VMEM((NP,), jnp.float32),
          pltpu.VMEM((span,), jnp.int32),
          pltpu.VMEM((span,), jnp.int32),
          pltpu.VMEM((NP,), jnp.float32),
      ],
  )
  def wsum_kernel(invout_hbm, src_hbm, dst_hbm, out_hbm,
                  inv_v, src_v, dst_v, hist_v):
    c = lax.axis_index("c")
    s = lax.axis_index("s")
    wid = c * NSUB + s
    zeros16 = jnp.zeros((16,), jnp.float32)

    def zero_body(k, carry):
      hist_v[pl.ds(k * 16, 16)] = zeros16
      return carry

    lax.fori_loop(0, NP // 16, zero_body, 0, unroll=8)

    pltpu.sync_copy(invout_hbm, inv_v)
    pltpu.sync_copy(src_hbm.at[wid], src_v)
    pltpu.sync_copy(dst_hbm.at[wid], dst_v)

    def hist_body(k, carry):
      si = src_v[pl.ds(k * 16, 16)]
      vals = plsc.load_gather(inv_v, [si])
      di = dst_v[pl.ds(k * 16, 16)]
      plsc.addupdate_scatter(hist_v, [di], vals)
      return carry

    lax.fori_loop(0, vecs, hist_body, 0, unroll=8)

    pltpu.sync_copy(hist_v, out_hbm.at[wid])

  return wsum_kernel


def _make_spmm(nch):
  ndep = 3
  assert nch % ndep == 0

  @functools.partial(
      pl.kernel,
      mesh=_mesh(),
      compiler_params=pltpu.CompilerParams(needs_layout_passes=False),
      out_type=jax.ShapeDtypeStruct((NSC, NP, H), jnp.float32),
      scratch_types=[
          pltpu.VMEM((ndep, 2, CHUNK), jnp.int32),
          [pltpu.VMEM((CHUNK, H), jnp.float32) for _ in range(ndep)],
          pltpu.VMEM_SHARED((NP, H), jnp.float32),
          [pltpu.SemaphoreType.DMA for _ in range(ndep)],
          [pltpu.SemaphoreType.DMA for _ in range(ndep)],
      ],
  )
  def spmm_kernel(hs_hbm, edge_hbm, out_hbm,
                  idxring, bufs, agg_sh, isems, dsems):
    c = lax.axis_index("c")
    s = lax.axis_index("s")
    wid = c * NSUB + s
    zeros16 = jnp.zeros((16,), jnp.float32)

    def idx_cp(k, slot):
      return pltpu.make_async_copy(edge_hbm.at[wid, k], idxring.at[slot],
                                   isems[slot])

    idx_cp(0, 0).start()
    idx_cp(1, 1).start()
    idx_cp(2, 2).start()

    def zb(k, carry):
      bufs[0][k // (H // 16), pl.ds((k % (H // 16)) * 16, 16)] = zeros16
      return carry

    lax.fori_loop(0, CHUNK * (H // 16), zb, 0, unroll=8)
    base = s * SUBROWS

    def zs(t, carry):
      pltpu.sync_copy(bufs[0].at[pl.ds(0, CHUNK)],
                      agg_sh.at[pl.ds(base + t * CHUNK, CHUNK)])
      return carry

    lax.fori_loop(0, SUBROWS // CHUNK, zs, 0)
    rem = SUBROWS % CHUNK
    if rem:
      pltpu.sync_copy(
          bufs[0].at[pl.ds(0, rem)],
          agg_sh.at[pl.ds(base + SUBROWS - rem, rem)])

    @pl.when(s == 0)
    def _():
      pltpu.sync_copy(bufs[0].at[pl.ds(0, TAIL)],
                      agg_sh.at[pl.ds(NSUB * SUBROWS, TAIL)])

    plsc.subcore_barrier()

    def gat_cp(slot):
      return pltpu.make_async_copy(hs_hbm.at[idxring.at[slot, 0]],
                                   bufs[slot], dsems[slot])

    for u in range(2):
      idx_cp(u, u).wait()
      gat_cp(u).start()

    def step(g, carry):
      for u in range(ndep):
        j = g * ndep + u
        gat_cp(u).wait()

        @pl.when(j + 2 < nch)
        def _():
          idx_cp(j + 2, (u + 2) % ndep).wait()
          gat_cp((u + 2) % ndep).start()

        pltpu.sync_copy(bufs[u], agg_sh.at[idxring.at[u, 1]], add=True)

        @pl.when(j + 3 < nch)
        def _():
          idx_cp(j + 3, u).start()
      return carry

    lax.fori_loop(0, nch // ndep, step, 0)
    plsc.subcore_barrier()

    pltpu.sync_copy(
        agg_sh.at[pl.ds(s * SUBROWS, SUBROWS)],
        out_hbm.at[c, pl.ds(s * SUBROWS, SUBROWS)])

    @pl.when(s == 0)
    def _():
      pltpu.sync_copy(
          agg_sh.at[pl.ds(NSUB * SUBROWS, TAIL)],
          out_hbm.at[c, pl.ds(NSUB * SUBROWS, TAIL)])

  return spmm_kernel


def _precompute_body(degp_ref, x_ref, wr_ref, wz_ref, wh_ref, bias_ref,
                     consts_ref, invout_ref, invin_ref):
  deg = jnp.sum(degp_ref[...], axis=1)
  inv = jnp.where(deg > 0, lax.rsqrt(deg), 0.0)
  invout_ref[...] = inv[0][:, None]
  invin_ref[...] = inv[1][:, None]

  x = x_ref[...]
  xr = jnp.dot(x, wr_ref[...], preferred_element_type=jnp.float32)
  xz = jnp.dot(x, wz_ref[...], preferred_element_type=jnp.float32)
  xh = jnp.dot(x, wh_ref[...], preferred_element_type=jnp.float32)
  proj = jnp.concatenate(
      [xr, xz, xh, jnp.zeros((5, H), jnp.float32)], axis=0)
  consts_ref[...] = proj + bias_ref[...]


def _precompute(degp, x, wr, wz, wh, bias_pack):
  return pl.pallas_call(
      _precompute_body,
      out_shape=[
          jax.ShapeDtypeStruct((8, H), jnp.float32),
          jax.ShapeDtypeStruct((NP, 1), jnp.float32),
          jax.ShapeDtypeStruct((NP, 1), jnp.float32),
      ],
  )(degp, x, wr, wz, wh, bias_pack)


def _gru_tail(agg, h_ref, invout_ref, c_ref, w_ref, hn_ref, hs_ref):
  gh = jnp.dot(agg, w_ref[...], preferred_element_type=jnp.float32)
  gh = gh + c_ref[3:4]
  r = jax.nn.sigmoid(c_ref[0:1] + gh)
  z = jax.nn.sigmoid(c_ref[1:2] + gh)
  ht = jnp.tanh(c_ref[2:3] + r * gh)
  hn = (1.0 - z) * h_ref[...] + z * ht
  hn_ref[...] = hn
  hs_ref[...] = hn * invout_ref[...]


def _step_body(p_ref, h_ref, invin_ref, invout_ref, c_ref, w_ref,
               hn_ref, hs_ref):
  agg = (p_ref[0] + p_ref[1]) * invin_ref[...]
  _gru_tail(agg, h_ref, invout_ref, c_ref, w_ref, hn_ref, hs_ref)


def _step2_body(s_ref, h_ref, invin_ref, invout_ref, c_ref, w_ref,
                hn_ref, hs_ref):
  agg = h_ref[...] * (invin_ref[...] * s_ref[...])
  _gru_tail(agg, h_ref, invout_ref, c_ref, w_ref, hn_ref, hs_ref)


def _reduce_wsum_body(wsum_ref, s_ref):
  s_ref[...] = jnp.sum(wsum_ref[...], axis=0)[:, None]


def _reduce_wsum(wsump):
  return pl.pallas_call(
      _reduce_wsum_body,
      out_shape=jax.ShapeDtypeStruct((NP, 1), jnp.float32),
  )(wsump)


def _tc_step(p, h, invin, invout, consts, gcn_W):
  grid = (NP // _RB,)
  return pl.pallas_call(
      _step_body,
      grid=grid,
      in_specs=[
          pl.BlockSpec((NSC, _RB, H), lambda j: (0, j, 0)),
          pl.BlockSpec((_RB, H), lambda j: (j, 0)),
          pl.BlockSpec((_RB, 1), lambda j: (j, 0)),
          pl.BlockSpec((_RB, 1), lambda j: (j, 0)),
          pl.BlockSpec((8, H), lambda j: (0, 0)),
          pl.BlockSpec((H, H), lambda j: (0, 0)),
      ],
      out_specs=[
          pl.BlockSpec((_RB, H), lambda j: (j, 0)),
          pl.BlockSpec((_RB, H), lambda j: (j, 0)),
      ],
      out_shape=[
          jax.ShapeDtypeStruct((NP, H), jnp.float32),
          jax.ShapeDtypeStruct((NP, H), jnp.float32),
      ],
  )(p, h, invin, invout, consts, gcn_W)


def _tc_step2(svec, h, invin, invout, consts, gcn_W):
  grid = (NP // _RB,)
  return pl.pallas_call(
      _step2_body,
      grid=grid,
      in_specs=[
          pl.BlockSpec((_RB, 1), lambda j: (j, 0)),
          pl.BlockSpec((_RB, H), lambda j: (j, 0)),
          pl.BlockSpec((_RB, 1), lambda j: (j, 0)),
          pl.BlockSpec((_RB, 1), lambda j: (j, 0)),
          pl.BlockSpec((8, H), lambda j: (0, 0)),
          pl.BlockSpec((H, H), lambda j: (0, 0)),
      ],
      out_specs=[
          pl.BlockSpec((_RB, H), lambda j: (j, 0)),
          pl.BlockSpec((_RB, H), lambda j: (j, 0)),
      ],
      out_shape=[
          jax.ShapeDtypeStruct((NP, H), jnp.float32),
          jax.ShapeDtypeStruct((NP, H), jnp.float32),
      ],
  )(svec, h, invin, invout, consts, gcn_W)


def kernel(x, edge_index, w_r_W, w_r_b, w_z_W, w_z_b, w_h_W, w_h_b,
           gcn_W, gcn_b):
  E = edge_index.shape[1]
  span = -(-E // NT)
  span = -(-span // (3 * CHUNK)) * (3 * CHUNK)
  ep = NT * span
  pad = ep - E
  nch = span // CHUNK

  src = edge_index[0]
  dst = edge_index[1]
  if pad:
    fill = N + (jnp.arange(pad, dtype=jnp.int32) % NDUM)
    src = jnp.concatenate([src, fill])
    dst = jnp.concatenate([dst, fill])
  src_f = src.reshape(NT, span)
  dst_f = dst.reshape(NT, span)
  src_t = src.reshape(NT, nch, CHUNK)
  dst_t = dst.reshape(NT, nch, CHUNK)
  edge_t = jnp.stack([src_t, dst_t], axis=2)

  degp = _make_degrees(span)(src_f, dst_f)

  bias_pack = jnp.zeros((8, H), jnp.float32)
  bias_pack = bias_pack.at[0].set(w_r_b).at[1].set(w_z_b)
  bias_pack = bias_pack.at[2].set(w_h_b).at[3].set(gcn_b)

  consts, invout, invin = _precompute(
      degp, x.reshape(1, H), w_r_W, w_z_W, w_h_W, bias_pack)

  svec = _reduce_wsum(_make_wsum(span)(invout.reshape(NP), src_f, dst_f))

  spmm = _make_spmm(nch)
  h = jnp.zeros((NP, H), jnp.float32)
  out = jnp.zeros((SEQ, N, H), jnp.float32)
  h, _ = _tc_step2(jnp.zeros((NP, 1), jnp.float32), h,
                   invin, invout, consts, gcn_W)
  out = out.at[0].set(h[:N])
  h, hs = _tc_step2(svec, h, invin, invout, consts, gcn_W)
  out = out.at[1].set(h[:N])
  for t in range(2, SEQ):
    p = spmm(hs, edge_t)
    h, hs = _tc_step(p, h, invin, invout, consts, gcn_W)
    out = out.at[t].set(h[:N])
  return out[None]
